# Initial kernel scaffold; baseline (speedup 1.0000x reference)
import jax
import jax.numpy as jnp
from jax.experimental import pallas as pl


def kernel(x, ei, W1l, b1l, W1r, gamma, beta, rm, rv, W2l, b2l, W2r):
    raise NotImplementedError("write your pallas kernel here")


# speedup vs baseline: 8.1521x; 8.1521x over previous
import functools

import jax
import jax.numpy as jnp
from jax import lax
from jax.experimental import pallas as pl
from jax.experimental.pallas import tpu as pltpu
from jax.experimental.pallas import tpu_sc as plsc

N = 10000
E = 320000
F = 128

NC = 2
NS = 16
NW = NC * NS
ET = E // NW
CH = 200
ONES = 208
NCHUNK = ET // CH
RT = 632
RTL = N - 15 * RT
CT = 2000


def _sc_agg_body(with_cnt, feat, src, dst, agg_out, cnt_out,
                 agg_sh, cnt_sh, sidx, didx, rows, ones, zv, sem):
    c = lax.axis_index("c")
    s = lax.axis_index("s")
    ebase = (c * NS + s) * ET

    def _zrow(i, carry):
        for j in range(F // 16):
            rows[i, pl.ds(j * 16, 16)] = jnp.zeros((16,), jnp.float32)
        return carry

    lax.fori_loop(0, CH, _zrow, 0)
    ZR = N // NS
    for k in range(ZR // CH):
        pltpu.sync_copy(rows, agg_sh.at[pl.ds(s * ZR + k * CH, CH)])
    if ZR % CH:
        pltpu.sync_copy(rows.at[pl.ds(0, ZR % CH)],
                        agg_sh.at[pl.ds(s * ZR + (ZR // CH) * CH, ZR % CH)])
    if with_cnt:
        def _zcnt(i, carry):
            zv[pl.ds(i * 16, 16)] = jnp.zeros((16,), jnp.float32)
            return carry

        lax.fori_loop(0, CT // 16, _zcnt, 0)

        @pl.when(s < N // CT)
        def _():
            pltpu.sync_copy(zv, cnt_sh.at[pl.ds(s * CT, CT)])
        for i in range(ONES // 16):
            ones[pl.ds(i * 16, 16)] = jnp.ones((16,), jnp.float32)
    plsc.subcore_barrier()

    def chunk(i, carry):
        base = ebase + i * CH
        pltpu.sync_copy(src.at[pl.ds(base, CH)], sidx)
        pltpu.sync_copy(dst.at[pl.ds(base, CH)], didx)
        pltpu.async_copy(feat.at[sidx], rows, sem).wait()
        pltpu.sync_copy(rows, agg_sh.at[didx], add=True)
        if with_cnt:
            pltpu.sync_copy(ones.at[pl.ds(0, CH)], cnt_sh.at[didx], add=True)
        return carry

    lax.fori_loop(0, NCHUNK, chunk, 0)
    plsc.subcore_barrier()

    @pl.when(s < NS - 1)
    def _():
        pltpu.sync_copy(agg_sh.at[pl.ds(s * RT, RT)],
                        agg_out.at[pl.ds(c * N + s * RT, RT)])

    @pl.when(s == NS - 1)
    def _():
        pltpu.sync_copy(agg_sh.at[pl.ds(s * RT, RTL)],
                        agg_out.at[pl.ds(c * N + s * RT, RTL)])
    if with_cnt:
        @pl.when(s < N // CT)
        def _():
            pltpu.sync_copy(cnt_sh.at[pl.ds(s * CT, CT)], zv)
            pltpu.sync_copy(zv, cnt_out.at[pl.ds(c * N + s * CT, CT)])


def _make_sc_agg(with_cnt):
    mesh = plsc.VectorSubcoreMesh(core_axis_name="c", subcore_axis_name="s",
                                  num_cores=NC, num_subcores=NS)
    return pl.kernel(
        functools.partial(_sc_agg_body, with_cnt),
        out_type=(
            jax.ShapeDtypeStruct((NC * N, F), jnp.float32),
            jax.ShapeDtypeStruct((NC * N,), jnp.float32),
        ),
        mesh=mesh,
        scratch_types=[
            pltpu.VMEM_SHARED((N, F), jnp.float32),
            pltpu.VMEM_SHARED((N,), jnp.float32),
            pltpu.VMEM((CH,), jnp.int32),
            pltpu.VMEM((CH,), jnp.int32),
            pltpu.VMEM((CH, F), jnp.float32),
            pltpu.VMEM((ONES,), jnp.float32),
            pltpu.VMEM((CT,), jnp.float32),
            pltpu.SemaphoreType.DMA,
        ],
        name="sage_sc_agg" + ("_cnt" if with_cnt else ""),
    )


_sc_agg_cnt = _make_sc_agg(True)
_sc_agg = _make_sc_agg(False)

BR = 2000


def _tc1_body(agg_ref, cnt_ref, x_ref, wl_ref, bl_ref, wr_ref,
              gm_ref, bt_ref, rm_ref, rv_ref, o_ref):
    agg = agg_ref[0] + agg_ref[1]
    cnt = cnt_ref[0] + cnt_ref[1]
    rinv = 1.0 / jnp.maximum(cnt, 1.0)
    z = (jnp.dot(agg * rinv, wl_ref[...], precision=lax.Precision.HIGHEST,
                 preferred_element_type=jnp.float32)
         + jnp.dot(x_ref[...], wr_ref[...], precision=lax.Precision.HIGHEST,
                   preferred_element_type=jnp.float32)
         + bl_ref[...])
    sc = gm_ref[...] * lax.rsqrt(rv_ref[...] + 1e-5)
    sh = bt_ref[...] - rm_ref[...] * sc
    o_ref[...] = jnp.maximum(z * sc + sh, 0.0)


def _tc2_body(agg_ref, cnt_ref, h_ref, wl_ref, bl_ref, wr_ref, o_ref):
    agg = agg_ref[0] + agg_ref[1]
    cnt = cnt_ref[0] + cnt_ref[1]
    rinv = 1.0 / jnp.maximum(cnt, 1.0)
    o_ref[...] = (jnp.dot(agg * rinv, wl_ref[...],
                          precision=lax.Precision.HIGHEST,
                          preferred_element_type=jnp.float32)
                  + jnp.dot(h_ref[...], wr_ref[...],
                            precision=lax.Precision.HIGHEST,
                            preferred_element_type=jnp.float32)
                  + bl_ref[...])


_row_spec = pl.BlockSpec((BR, F), lambda i: (i, 0))
_agg_spec = pl.BlockSpec((NC, BR, F), lambda i: (0, i, 0))
_cnt_spec = pl.BlockSpec((NC, BR, 1), lambda i: (0, i, 0))
_vec_spec = pl.BlockSpec((1, F), lambda i: (0, 0))


def _tc1(agg, cnt, x, wl, bl, wr, gm, bt, rm, rv):
    return pl.pallas_call(
        _tc1_body,
        grid=(N // BR,),
        in_specs=[_agg_spec, _cnt_spec, _row_spec] + [_vec_spec] * 0 +
                 [pl.BlockSpec((F, F), lambda i: (0, 0)), _vec_spec,
                  pl.BlockSpec((F, F), lambda i: (0, 0)),
                  _vec_spec, _vec_spec, _vec_spec, _vec_spec],
        out_specs=_row_spec,
        out_shape=jax.ShapeDtypeStruct((N, F), jnp.float32),
    )(agg, cnt, x, wl, bl, wr, gm, bt, rm, rv)


def _tc2(agg, cnt, h, wl, bl, wr):
    return pl.pallas_call(
        _tc2_body,
        grid=(N // BR,),
        in_specs=[_agg_spec, _cnt_spec, _row_spec,
                  pl.BlockSpec((F, F), lambda i: (0, 0)), _vec_spec,
                  pl.BlockSpec((F, F), lambda i: (0, 0))],
        out_specs=_row_spec,
        out_shape=jax.ShapeDtypeStruct((N, F), jnp.float32),
    )(agg, cnt, h, wl, bl, wr)


def kernel(x, ei, W1l, b1l, W1r, gamma, beta, rm, rv, W2l, b2l, W2r):
    src = ei[0]
    dst = ei[1]

    agg1, cnt = _sc_agg_cnt(x, src, dst)
    agg1 = agg1.reshape(NC, N, F)
    cnt3 = cnt.reshape(NC, N, 1)
    h = _tc1(agg1, cnt3, x, W1l, b1l.reshape(1, F), W1r,
             gamma.reshape(1, F), beta.reshape(1, F),
             rm.reshape(1, F), rv.reshape(1, F))

    agg2, _ = _sc_agg(h, src, dst)
    agg2 = agg2.reshape(NC, N, F)
    out = _tc2(agg2, cnt3, h, W2l, b2l.reshape(1, F), W2r)
    return out
